# Initial kernel scaffold; baseline (speedup 1.0000x reference)
import functools

import jax
import jax.numpy as jnp
from jax import lax
from jax.experimental import pallas as pl
from jax.experimental.pallas import tpu as pltpu
from jax.experimental.pallas import tpu_sc as plsc

NC = 2
NS = 16
NW = NC * NS
LANES = 16


def _gemm_body(x_ref, w_ref, o_ref):
    o_ref[...] = jnp.dot(x_ref[...], w_ref[0],
                         preferred_element_type=jnp.float32)


def _relation_gemm(x, weight, bn):
    n, d_in = x.shape
    r, _, d_out = weight.shape
    return pl.pallas_call(
        _gemm_body,
        grid=(r, n // bn),
        in_specs=[
            pl.BlockSpec((bn, d_in), lambda i, j: (j, 0)),
            pl.BlockSpec((1, d_in, d_out), lambda i, j: (i, 0, 0)),
        ],
        out_specs=pl.BlockSpec((1, bn, d_out), lambda i, j: (i, j, 0)),
        out_shape=jax.ShapeDtypeStruct((r, n, d_out), jnp.float32),
    )(x, weight)


def _add_body(p_ref, o_ref):
    o_ref[...] = p_ref[0] + p_ref[1]


def _pair_add(p, bn):
    _, n, d = p.shape
    return pl.pallas_call(
        _add_body,
        grid=(n // bn,),
        in_specs=[pl.BlockSpec((2, bn, d), lambda i: (0, i, 0))],
        out_specs=pl.BlockSpec((bn, d), lambda i: (i, 0)),
        out_shape=jax.ShapeDtypeStruct((n, d), jnp.float32),
    )(p)


def _make_sc_scatter(n_nodes, d, n_edges):
    ept = n_edges // NW
    b = 80
    nch = ept // b
    rpt = n_nodes // NS
    assert ept * NW == n_edges and nch * b == ept and rpt * NS == n_nodes
    assert b % LANES == 0 and ept % 8 == 0

    mesh = plsc.VectorSubcoreMesh(core_axis_name="c", subcore_axis_name="s")

    @functools.partial(
        pl.kernel,
        out_type=jax.ShapeDtypeStruct((NC, n_nodes, d), jnp.float32),
        mesh=mesh,
        scratch_types=[
            pltpu.VMEM((ept,), jnp.int32),
            pltpu.VMEM((ept,), jnp.int32),
            pltpu.VMEM((ept,), jnp.int32),
            pltpu.VMEM((nch, b), jnp.int32),
            pltpu.VMEM((nch, b), jnp.int32),
            pltpu.VMEM((2, b, d), jnp.float32),
            pltpu.VMEM_SHARED((n_nodes, d), jnp.float32),
            pltpu.SemaphoreType.DMA,
            pltpu.SemaphoreType.DMA,
            pltpu.SemaphoreType.DMA,
        ],
    )
    def sc_scatter(y_hbm, src_hbm, dst_hbm, rel_hbm, zeros_hbm, out_hbm,
                   src_v, rel_v, dst_v, gid2, did2, rows, h_sh,
                   sem_i, sem_a, sem_b):
        cid = lax.axis_index("c")
        sid = lax.axis_index("s")
        wid = cid * NS + sid
        base = wid * ept

        cp_s = pltpu.async_copy(src_hbm.at[pl.ds(base, ept)], src_v, sem_i)
        cp_r = pltpu.async_copy(rel_hbm.at[pl.ds(base, ept)], rel_v, sem_i)
        cp_d = pltpu.async_copy(dst_hbm.at[pl.ds(base, ept)], dst_v, sem_i)

        pltpu.sync_copy(zeros_hbm.at[pl.ds(sid * rpt, rpt)],
                        h_sh.at[pl.ds(sid * rpt, rpt)])

        cp_s.wait()
        cp_r.wait()
        cp_d.wait()

        vpr = b // LANES

        def build(ci, carry):
            for j in range(vpr):
                o = ci * b + j * LANES
                s = src_v[pl.ds(o, LANES)]
                r = rel_v[pl.ds(o, LANES)]
                t = dst_v[pl.ds(o, LANES)]
                gid2[ci, pl.ds(j * LANES, LANES)] = r * n_nodes + s
                did2[ci, pl.ds(j * LANES, LANES)] = t
            return carry

        lax.fori_loop(0, nch, build, 0)

        plsc.subcore_barrier()

        pltpu.async_copy(y_hbm.at[gid2.at[0]], rows.at[0], sem_a)

        def step(t, carry):
            c0 = 2 * t
            c1 = 2 * t + 1
            pltpu.make_async_copy(y_hbm.at[gid2.at[c0]], rows.at[0], sem_a).wait()
            pltpu.async_copy(y_hbm.at[gid2.at[c1]], rows.at[1], sem_b)
            pltpu.sync_copy(rows.at[0], h_sh.at[did2.at[c0]], add=True)
            pltpu.make_async_copy(y_hbm.at[gid2.at[c1]], rows.at[1], sem_b).wait()
            pltpu.async_copy(y_hbm.at[gid2.at[c1 + 1]], rows.at[0], sem_a)
            pltpu.sync_copy(rows.at[1], h_sh.at[did2.at[c1]], add=True)
            return carry

        lax.fori_loop(0, (nch - 1) // 2, step, 0)
        pltpu.make_async_copy(y_hbm.at[gid2.at[nch - 1]], rows.at[0], sem_a).wait()
        pltpu.sync_copy(rows.at[0], h_sh.at[did2.at[nch - 1]], add=True)

        plsc.subcore_barrier()
        pltpu.sync_copy(h_sh.at[pl.ds(sid * rpt, rpt)],
                        out_hbm.at[cid, pl.ds(sid * rpt, rpt)])

    return sc_scatter


def kernel(x, edge_index, rel_type, weight):
    n, _ = x.shape
    r, _, d_out = weight.shape
    e = edge_index.shape[1]
    src = edge_index[0]
    dst = edge_index[1]
    y = _relation_gemm(x, weight, 1000).reshape(r * n, d_out)
    zeros = jnp.zeros((n, d_out), jnp.float32)
    partials = _make_sc_scatter(n, d_out, e)(y, src, dst, rel_type, zeros)
    return _pair_add(partials, 1000)


# speedup vs baseline: 12.7838x; 12.7838x over previous
import functools

import jax
import jax.numpy as jnp
from jax import lax
from jax.experimental import pallas as pl
from jax.experimental.pallas import tpu as pltpu
from jax.experimental.pallas import tpu_sc as plsc

NC = 2
NS = 16
NW = NC * NS


def _gemm_body(x_ref, w_ref, o_ref):
    o_ref[0] = jnp.dot(x_ref[...], w_ref[0],
                       preferred_element_type=jnp.float32)


def _relation_gemm(x, weight, bn):
    n, d_in = x.shape
    r, _, d_out = weight.shape
    return pl.pallas_call(
        _gemm_body,
        grid=(r, n // bn),
        in_specs=[
            pl.BlockSpec((bn, d_in), lambda i, j: (j, 0)),
            pl.BlockSpec((1, d_in, d_out), lambda i, j: (i, 0, 0)),
        ],
        out_specs=pl.BlockSpec((1, bn, d_out), lambda i, j: (i, j, 0)),
        out_shape=jax.ShapeDtypeStruct((r, n, d_out), jnp.float32),
    )(x, weight)


def _gid_body(n_nodes, s_ref, r_ref, o_ref):
    o_ref[...] = r_ref[...] * n_nodes + s_ref[...]


def _edge_gid(src, rel, n_nodes):
    e = src.shape[0]
    s2 = src.reshape(e // 128, 128)
    r2 = rel.reshape(e // 128, 128)
    out = pl.pallas_call(
        functools.partial(_gid_body, n_nodes),
        out_shape=jax.ShapeDtypeStruct(s2.shape, jnp.int32),
    )(s2, r2)
    return out.reshape(e)


def _add_body(p_ref, o_ref):
    o_ref[...] = p_ref[0] + p_ref[1]


def _pair_add(p, bn):
    _, n, d = p.shape
    return pl.pallas_call(
        _add_body,
        grid=(n // bn,),
        in_specs=[pl.BlockSpec((2, bn, d), lambda i: (0, i, 0))],
        out_specs=pl.BlockSpec((bn, d), lambda i: (i, 0)),
        out_shape=jax.ShapeDtypeStruct((n, d), jnp.float32),
    )(p)


def _make_sc_scatter(n_nodes, d, n_edges):
    ept = n_edges // NW
    b = 80
    nch = ept // b
    rpt_a = (n_nodes // NS) & ~7
    rpt_z = n_nodes - rpt_a * (NS - 1)
    assert ept * NW == n_edges and nch * b == ept and nch % 2 == 1
    assert b % 8 == 0 and ept % 8 == 0 and rpt_a % 8 == 0

    mesh = plsc.VectorSubcoreMesh(core_axis_name="c", subcore_axis_name="s",
                                  num_cores=NC, num_subcores=NS)

    @functools.partial(
        pl.kernel,
        out_type=jax.ShapeDtypeStruct((NC, n_nodes, d), jnp.float32),
        mesh=mesh,
        scratch_types=[
            pltpu.VMEM((ept,), jnp.int32),
            pltpu.VMEM((nch, b), jnp.int32),
            pltpu.VMEM((2, b, d), jnp.float32),
            pltpu.VMEM_SHARED((n_nodes, d), jnp.float32),
            pltpu.SemaphoreType.DMA,
            pltpu.SemaphoreType.DMA,
            pltpu.SemaphoreType.DMA,
            pltpu.SemaphoreType.DMA,
        ],
    )
    def sc_scatter(y_hbm, g_hbm, dst_hbm, zeros_hbm, out_hbm,
                   gid, did2, rows, h_sh, sem_i, sem_d, sem_a, sem_b):
        cid = lax.axis_index("c")
        sid = lax.axis_index("s")
        wid = cid * NS + sid
        base = wid * ept

        cp_g = pltpu.async_copy(g_hbm.at[pl.ds(base, ept)], gid, sem_i)

        def fill_did(c, carry):
            pltpu.async_copy(dst_hbm.at[pl.ds(base + c * b, b)],
                             did2.at[c], sem_d)
            return carry

        lax.fori_loop(0, nch, fill_did, 0)

        @pl.when(sid < NS - 1)
        def _():
            pltpu.sync_copy(zeros_hbm.at[pl.ds(sid * rpt_a, rpt_a)],
                            h_sh.at[pl.ds(sid * rpt_a, rpt_a)])

        @pl.when(sid == NS - 1)
        def _():
            pltpu.sync_copy(zeros_hbm.at[pl.ds(rpt_a * (NS - 1), rpt_z)],
                            h_sh.at[pl.ds(rpt_a * (NS - 1), rpt_z)])

        cp_g.wait()

        def drain_did(c, carry):
            pltpu.make_async_copy(dst_hbm.at[pl.ds(base, b)],
                                  did2.at[0], sem_d).wait()
            return carry

        lax.fori_loop(0, nch, drain_did, 0)

        plsc.subcore_barrier()

        pltpu.async_copy(y_hbm.at[gid.at[pl.ds(0, b)]], rows.at[0], sem_a)

        def step(t, carry):
            c0 = 2 * t
            c1 = 2 * t + 1
            pltpu.make_async_copy(y_hbm.at[gid.at[pl.ds(c0 * b, b)]],
                                  rows.at[0], sem_a).wait()
            pltpu.async_copy(y_hbm.at[gid.at[pl.ds(c1 * b, b)]],
                             rows.at[1], sem_b)
            pltpu.sync_copy(rows.at[0], h_sh.at[did2.at[c0]], add=True)
            pltpu.make_async_copy(y_hbm.at[gid.at[pl.ds(c1 * b, b)]],
                                  rows.at[1], sem_b).wait()
            pltpu.async_copy(y_hbm.at[gid.at[pl.ds((c1 + 1) * b, b)]],
                             rows.at[0], sem_a)
            pltpu.sync_copy(rows.at[1], h_sh.at[did2.at[c1]], add=True)
            return carry

        lax.fori_loop(0, (nch - 1) // 2, step, 0)
        pltpu.make_async_copy(y_hbm.at[gid.at[pl.ds((nch - 1) * b, b)]],
                              rows.at[0], sem_a).wait()
        pltpu.sync_copy(rows.at[0], h_sh.at[did2.at[nch - 1]], add=True)

        plsc.subcore_barrier()

        @pl.when(sid < NS - 1)
        def _():
            pltpu.sync_copy(h_sh.at[pl.ds(sid * rpt_a, rpt_a)],
                            out_hbm.at[cid, pl.ds(sid * rpt_a, rpt_a)])

        @pl.when(sid == NS - 1)
        def _():
            pltpu.sync_copy(h_sh.at[pl.ds(rpt_a * (NS - 1), rpt_z)],
                            out_hbm.at[cid, pl.ds(rpt_a * (NS - 1), rpt_z)])

    return sc_scatter


def kernel(x, edge_index, rel_type, weight):
    n, _ = x.shape
    r, _, d_out = weight.shape
    e = edge_index.shape[1]
    src = edge_index[0]
    dst = edge_index[1]
    y = _relation_gemm(x, weight, 1000).reshape(r * n, d_out)
    g = _edge_gid(src, rel_type, n)
    zeros = jnp.zeros((n, d_out), jnp.float32)
    partials = _make_sc_scatter(n, d_out, e)(y, g, dst, zeros)
    return _pair_add(partials, 1000)
